# Initial kernel scaffold; baseline (speedup 1.0000x reference)
import jax
import jax.numpy as jnp
from jax.experimental import pallas as pl


def kernel(emb, wt1, bt1, wt2, bt2, w1, b1, w2, b2, w3, b3, w4, b4, x, t, y):
    raise NotImplementedError("write your pallas kernel here")


# speedup vs baseline: 1.8680x; 1.8680x over previous
import jax
import jax.numpy as jnp
from jax.experimental import pallas as pl
from jax.experimental.pallas import tpu as pltpu

_LATENT = 10
_NUM_CLASSES = 10
_TIME_EMB = 32
_TIMESTEPS = 300
_SLAB_K = 128
_T_OFF = 0
_X_OFF = _TIME_EMB
_Y_OFF = _TIME_EMB + _LATENT


def _round_up(n, m):
    return ((n + m - 1) // m) * m


def _mlp_kernel(x_ref, tn_ref, y_ref,
                wt1_ref, bt1_ref,
                w1p_ref, b1_ref, w2_ref, b2_ref,
                w3_ref, b3_ref, w4_ref, b4_ref,
                out_ref, slab_ref):
    f32 = jnp.float32
    bf16 = jnp.bfloat16
    bm = x_ref.shape[0]
    d_x = x_ref.shape[1]

    h_t = jnp.maximum(tn_ref[...] * wt1_ref[...] + bt1_ref[...], 0.0)

    lane = jax.lax.broadcasted_iota(jnp.int32, (bm, _SLAB_K), 1)
    slab_ref[...] = (lane == (y_ref[...] + _Y_OFF)).astype(f32)
    slab_ref[:, _T_OFF:_T_OFF + _TIME_EMB] = h_t
    slab_ref[:, _X_OFF:_X_OFF + d_x] = x_ref[...]

    h1 = jnp.maximum(
        jnp.dot(slab_ref[...].astype(bf16), w1p_ref[...],
                preferred_element_type=f32) + b1_ref[...], 0.0)
    h2 = jnp.maximum(
        jnp.dot(h1.astype(bf16), w2_ref[...],
                preferred_element_type=f32) + b2_ref[...], 0.0)
    h3 = jnp.maximum(
        jnp.dot(h2.astype(bf16), w3_ref[...],
                preferred_element_type=f32) + b3_ref[...], 0.0)
    out_ref[...] = jnp.dot(h3.astype(bf16), w4_ref[...],
                           preferred_element_type=f32) + b4_ref[...]


def kernel(emb, wt1, bt1, wt2, bt2, w1, b1, w2, b2, w3, b3, w4, b4, x, t, y):
    f32 = jnp.float32
    bf16 = jnp.bfloat16
    B, latent_dim = x.shape

    block_m = 1024 if B >= 1024 else max(8, _round_up(B, 8))
    Bp = _round_up(B, block_m)

    t_norm = (t.astype(f32) / _TIMESTEPS).reshape(B, 1)
    y2d = y.astype(jnp.int32).reshape(B, 1)
    if Bp != B:
        pad = Bp - B
        x = jnp.pad(x, ((0, pad), (0, 0)))
        t_norm = jnp.pad(t_norm, ((0, pad), (0, 0)))
        y2d = jnp.pad(y2d, ((0, pad), (0, 0)))

    w1x = w1[:latent_dim]
    w1t = w1[latent_dim:latent_dim + _TIME_EMB]
    w1y = w1[latent_dim + _TIME_EMB:]
    w1t_folded = jnp.dot(wt2, w1t, preferred_element_type=f32)
    w1y_folded = jnp.dot(emb, w1y, preferred_element_type=f32)
    b1_folded = b1 + jnp.dot(bt2, w1t, preferred_element_type=f32)
    w1p = jnp.zeros((_SLAB_K, w1.shape[1]), f32)
    w1p = w1p.at[_T_OFF:_T_OFF + _TIME_EMB].set(w1t_folded)
    w1p = w1p.at[_X_OFF:_X_OFF + latent_dim].set(w1x)
    w1p = w1p.at[_Y_OFF:_Y_OFF + _NUM_CLASSES].set(w1y_folded)

    weights = (wt1, bt1,
               w1p.astype(bf16), b1_folded,
               w2.astype(bf16), b2,
               w3.astype(bf16), b3,
               w4.astype(bf16), b4)

    VMEM = pltpu.MemorySpace.VMEM

    def act_spec(d):
        return pl.BlockSpec((block_m, d), lambda i: (i, 0), memory_space=VMEM)

    def const_spec(shape):
        return pl.BlockSpec(shape, lambda i: (0, 0), memory_space=VMEM)

    in_specs = [act_spec(latent_dim), act_spec(1), act_spec(1)]
    in_specs += [const_spec(w.shape) for w in weights]

    out = pl.pallas_call(
        _mlp_kernel,
        out_shape=jax.ShapeDtypeStruct((Bp, latent_dim), f32),
        grid=(Bp // block_m,),
        in_specs=in_specs,
        out_specs=act_spec(latent_dim),
        scratch_shapes=[pltpu.VMEM((block_m, _SLAB_K), f32)],
        compiler_params=pltpu.CompilerParams(
            dimension_semantics=("parallel",)),
    )(x, t_norm, y2d, *weights)
    return out[:B]


# speedup vs baseline: 2.1200x; 1.1349x over previous
import jax
import jax.numpy as jnp
from jax.experimental import pallas as pl
from jax.experimental.pallas import tpu as pltpu

_LATENT = 10
_NUM_CLASSES = 10
_TIME_EMB = 32
_TIMESTEPS = 300
_SLAB_K = 128
_X_OFF = 0
_T_OFF = 32
_Y_OFF = 64


def _round_up(n, m):
    return ((n + m - 1) // m) * m


def _mlp_kernel(x_ref, c_ref,
                wt1p_ref, bt1p_ref,
                w1p_ref, b1_ref, w2_ref, b2_ref,
                w3_ref, b3_ref, w4_ref, b4_ref,
                out_ref, slab_ref):
    f32 = jnp.float32
    bf16 = jnp.bfloat16
    bm = x_ref.shape[0]
    d_x = x_ref.shape[1]

    eye2 = (jax.lax.broadcasted_iota(jnp.int32, (2, 2), 0) ==
            jax.lax.broadcasted_iota(jnp.int32, (2, 2), 1)).astype(f32)
    ct = jax.lax.dot_general(c_ref[...], eye2, (((0,), (0,)), ((), ())),
                             preferred_element_type=f32)
    tn_col = ct[:, 0:1]
    y_col = ct[:, 1:2].astype(jnp.int32)

    h_full = jnp.maximum(tn_col * wt1p_ref[...] + bt1p_ref[...], 0.0)
    lane = jax.lax.broadcasted_iota(jnp.int32, (bm, _SLAB_K), 1)
    onehot = (lane == y_col + _Y_OFF).astype(f32)
    slab_ref[...] = (h_full + onehot).astype(bf16)
    slab_ref[:, _X_OFF:_X_OFF + d_x] = x_ref[...].astype(bf16)

    h1 = jnp.maximum(
        jnp.dot(slab_ref[...], w1p_ref[...],
                preferred_element_type=f32) + b1_ref[...], 0.0)
    h2 = jnp.maximum(
        jnp.dot(h1.astype(bf16), w2_ref[...],
                preferred_element_type=f32) + b2_ref[...], 0.0)
    h3 = jnp.maximum(
        jnp.dot(h2.astype(bf16), w3_ref[...],
                preferred_element_type=f32) + b3_ref[...], 0.0)
    out_ref[...] = jnp.dot(h3.astype(bf16), w4_ref[...],
                           preferred_element_type=f32) + b4_ref[...]


def kernel(emb, wt1, bt1, wt2, bt2, w1, b1, w2, b2, w3, b3, w4, b4, x, t, y):
    f32 = jnp.float32
    bf16 = jnp.bfloat16
    B, latent_dim = x.shape

    block_m = 1024 if B >= 1024 else max(8, _round_up(B, 8))
    Bp = _round_up(B, block_m)

    c = jnp.stack([t.astype(f32) / _TIMESTEPS, y.astype(f32)])
    if Bp != B:
        pad = Bp - B
        x = jnp.pad(x, ((0, pad), (0, 0)))
        c = jnp.pad(c, ((0, 0), (0, pad)))

    w1x = w1[:latent_dim]
    w1t = w1[latent_dim:latent_dim + _TIME_EMB]
    w1y = w1[latent_dim + _TIME_EMB:]
    w1t_folded = jnp.dot(wt2, w1t, preferred_element_type=f32)
    w1y_folded = jnp.dot(emb, w1y, preferred_element_type=f32)
    b1_folded = b1 + jnp.dot(bt2, w1t, preferred_element_type=f32)
    w1p = jnp.zeros((_SLAB_K, w1.shape[1]), f32)
    w1p = w1p.at[_X_OFF:_X_OFF + latent_dim].set(w1x)
    w1p = w1p.at[_T_OFF:_T_OFF + _TIME_EMB].set(w1t_folded)
    w1p = w1p.at[_Y_OFF:_Y_OFF + _NUM_CLASSES].set(w1y_folded)
    wt1p = jnp.zeros((1, _SLAB_K), f32).at[:, _T_OFF:_T_OFF + _TIME_EMB].set(wt1)
    bt1p = jnp.zeros((1, _SLAB_K), f32).at[:, _T_OFF:_T_OFF + _TIME_EMB].set(bt1)

    weights = (wt1p, bt1p,
               w1p.astype(bf16), b1_folded,
               w2.astype(bf16), b2,
               w3.astype(bf16), b3,
               w4.astype(bf16), b4)

    VMEM = pltpu.MemorySpace.VMEM

    def act_spec(d):
        return pl.BlockSpec((block_m, d), lambda i: (i, 0), memory_space=VMEM)

    def const_spec(shape):
        return pl.BlockSpec(shape, lambda i: (0, 0), memory_space=VMEM)

    in_specs = [act_spec(latent_dim),
                pl.BlockSpec((2, block_m), lambda i: (0, i), memory_space=VMEM)]
    in_specs += [const_spec(w.shape) for w in weights]

    out = pl.pallas_call(
        _mlp_kernel,
        out_shape=jax.ShapeDtypeStruct((Bp, latent_dim), f32),
        grid=(Bp // block_m,),
        in_specs=in_specs,
        out_specs=act_spec(latent_dim),
        scratch_shapes=[pltpu.VMEM((block_m, _SLAB_K), bf16)],
        compiler_params=pltpu.CompilerParams(
            dimension_semantics=("parallel",)),
    )(x, c, *weights)
    return out[:B]


# speedup vs baseline: 3.9122x; 1.8454x over previous
import jax
import jax.numpy as jnp
from jax.experimental import pallas as pl
from jax.experimental.pallas import tpu as pltpu

_LATENT = 10
_NUM_CLASSES = 10
_TIME_EMB = 32
_TIMESTEPS = 300
_SLAB_K = 64
_X_OFF = 0
_Y_OFF = 16
_T_OFF = 32
_C_ROWS = 16
_TN_ROW = 10
_Y_ROW = 11
_OUT_ROWS = 16


def _round_up(n, m):
    return ((n + m - 1) // m) * m


def _mlp_kernel(c_ref,
                wt1c_ref, bt1c_ref,
                w1p_ref, b1c_ref, w2_ref, b2c_ref,
                w3_ref, b3c_ref, w4p_ref, b4c_ref,
                out_ref, slab_ref):
    f32 = jnp.float32
    bf16 = jnp.bfloat16
    bn = c_ref.shape[1]

    tn_row = c_ref[_TN_ROW:_TN_ROW + 1, :]
    y_row = c_ref[_Y_ROW:_Y_ROW + 1, :].astype(jnp.int32)

    sub16 = jax.lax.broadcasted_iota(jnp.int32, (_C_ROWS, bn), 0)
    slab_ref[_X_OFF:_X_OFF + _C_ROWS, :] = jnp.where(
        sub16 < _LATENT, c_ref[...], 0.0).astype(bf16)
    slab_ref[_Y_OFF:_Y_OFF + _C_ROWS, :] = (sub16 == y_row).astype(bf16)
    slab_ref[_T_OFF:_T_OFF + _TIME_EMB, :] = jnp.maximum(
        wt1c_ref[...] * tn_row + bt1c_ref[...], 0.0).astype(bf16)

    dn = (((0,), (0,)), ((), ()))
    h1 = jnp.maximum(
        jax.lax.dot_general(w1p_ref[...], slab_ref[...], dn,
                            preferred_element_type=f32) + b1c_ref[...], 0.0)
    h2 = jnp.maximum(
        jax.lax.dot_general(w2_ref[...], h1.astype(bf16), dn,
                            preferred_element_type=f32) + b2c_ref[...], 0.0)
    h3 = jnp.maximum(
        jax.lax.dot_general(w3_ref[...], h2.astype(bf16), dn,
                            preferred_element_type=f32) + b3c_ref[...], 0.0)
    out_ref[...] = jax.lax.dot_general(w4p_ref[...], h3.astype(bf16), dn,
                                       preferred_element_type=f32) + b4c_ref[...]


def kernel(emb, wt1, bt1, wt2, bt2, w1, b1, w2, b2, w3, b3, w4, b4, x, t, y):
    f32 = jnp.float32
    bf16 = jnp.bfloat16
    B, latent_dim = x.shape

    block_n = 1024 if B >= 1024 else max(128, _round_up(B, 128))
    Bp = _round_up(B, block_n)

    c = jnp.concatenate([
        x.T,
        (t.astype(f32) / _TIMESTEPS).reshape(1, B),
        y.astype(f32).reshape(1, B),
        jnp.zeros((_C_ROWS - latent_dim - 2, B), f32),
    ])
    if Bp != B:
        c = jnp.pad(c, ((0, 0), (0, Bp - B)))

    w1x = w1[:latent_dim]
    w1t = w1[latent_dim:latent_dim + _TIME_EMB]
    w1y = w1[latent_dim + _TIME_EMB:]
    w1p = jnp.zeros((_SLAB_K, w1.shape[1]), f32)
    w1p = w1p.at[_X_OFF:_X_OFF + latent_dim].set(w1x)
    w1p = w1p.at[_Y_OFF:_Y_OFF + _NUM_CLASSES].set(
        jnp.dot(emb, w1y, preferred_element_type=f32))
    w1p = w1p.at[_T_OFF:_T_OFF + _TIME_EMB].set(
        jnp.dot(wt2, w1t, preferred_element_type=f32))
    b1c = (b1 + jnp.dot(bt2, w1t, preferred_element_type=f32)).reshape(-1, 1)
    w4p = jnp.zeros((w4.shape[0], _OUT_ROWS), f32).at[:, :latent_dim].set(w4)
    b4c = jnp.zeros((_OUT_ROWS, 1), f32).at[:latent_dim].set(b4.reshape(-1, 1))

    weights = (wt1.reshape(-1, 1), bt1.reshape(-1, 1),
               w1p.astype(bf16), b1c,
               w2.astype(bf16), b2.reshape(-1, 1),
               w3.astype(bf16), b3.reshape(-1, 1),
               w4p.astype(bf16), b4c)

    VMEM = pltpu.MemorySpace.VMEM

    def const_spec(shape):
        return pl.BlockSpec(shape, lambda i: (0, 0), memory_space=VMEM)

    in_specs = [pl.BlockSpec((_C_ROWS, block_n), lambda i: (0, i),
                             memory_space=VMEM)]
    in_specs += [const_spec(w.shape) for w in weights]

    out_t = pl.pallas_call(
        _mlp_kernel,
        out_shape=jax.ShapeDtypeStruct((_OUT_ROWS, Bp), f32),
        grid=(Bp // block_n,),
        in_specs=in_specs,
        out_specs=pl.BlockSpec((_OUT_ROWS, block_n), lambda i: (0, i),
                               memory_space=VMEM),
        scratch_shapes=[pltpu.VMEM((_SLAB_K, block_n), bf16)],
        compiler_params=pltpu.CompilerParams(
            dimension_semantics=("parallel",)),
    )(c, *weights)
    return out_t[:latent_dim, :B].T


# speedup vs baseline: 4.9202x; 1.2577x over previous
import jax
import jax.numpy as jnp
from jax.experimental import pallas as pl
from jax.experimental.pallas import tpu as pltpu

_LATENT = 10
_NUM_CLASSES = 10
_TIME_EMB = 32
_TIMESTEPS = 300
_SLAB_K = 64
_X_OFF = 0
_Y_OFF = 16
_T_OFF = 32
_C_ROWS = 16
_TN_ROW = 10
_Y_ROW = 11
_OUT_ROWS = 16


def _round_up(n, m):
    return ((n + m - 1) // m) * m


def _mlp_kernel(c_ref,
                wt1c_ref, bt1c_ref,
                w1p_ref, b1c_ref, w2_ref, b2c_ref,
                w3_ref, b3c_ref, w4p_ref, b4c_ref,
                out_ref, slab_ref):
    f32 = jnp.float32
    bf16 = jnp.bfloat16
    bn = c_ref.shape[1]

    tn_row = c_ref[_TN_ROW:_TN_ROW + 1, :]
    y_row = c_ref[_Y_ROW:_Y_ROW + 1, :].astype(jnp.int32)

    sub16 = jax.lax.broadcasted_iota(jnp.int32, (_C_ROWS, bn), 0)
    slab_ref[_X_OFF:_X_OFF + _C_ROWS, :] = jnp.where(
        sub16 < _LATENT, c_ref[...], 0.0).astype(bf16)
    slab_ref[_Y_OFF:_Y_OFF + _C_ROWS, :] = (sub16 == y_row).astype(bf16)
    slab_ref[_T_OFF:_T_OFF + _TIME_EMB, :] = jnp.maximum(
        wt1c_ref[...] * tn_row + bt1c_ref[...], 0.0).astype(bf16)

    dn = (((0,), (0,)), ((), ()))
    h1 = jnp.maximum(
        jax.lax.dot_general(w1p_ref[...], slab_ref[...], dn,
                            preferred_element_type=f32) + b1c_ref[...], 0.0)
    h2 = jnp.maximum(
        jax.lax.dot_general(w2_ref[...], h1.astype(bf16), dn,
                            preferred_element_type=f32) + b2c_ref[...], 0.0)
    h3 = jnp.maximum(
        jax.lax.dot_general(w3_ref[...], h2.astype(bf16), dn,
                            preferred_element_type=f32) + b3c_ref[...], 0.0)
    out_ref[...] = jax.lax.dot_general(w4p_ref[...], h3.astype(bf16), dn,
                                       preferred_element_type=f32) + b4c_ref[...]


def kernel(emb, wt1, bt1, wt2, bt2, w1, b1, w2, b2, w3, b3, w4, b4, x, t, y):
    f32 = jnp.float32
    bf16 = jnp.bfloat16
    B, latent_dim = x.shape

    block_n = 2048 if B >= 2048 else max(128, _round_up(B, 128))
    Bp = _round_up(B, block_n)

    c = jnp.concatenate([
        x.T,
        (t.astype(f32) / _TIMESTEPS).reshape(1, B),
        y.astype(f32).reshape(1, B),
        jnp.zeros((_C_ROWS - latent_dim - 2, B), f32),
    ])
    if Bp != B:
        c = jnp.pad(c, ((0, 0), (0, Bp - B)))

    w1x = w1[:latent_dim]
    w1t = w1[latent_dim:latent_dim + _TIME_EMB]
    w1y = w1[latent_dim + _TIME_EMB:]
    w1p = jnp.zeros((_SLAB_K, w1.shape[1]), f32)
    w1p = w1p.at[_X_OFF:_X_OFF + latent_dim].set(w1x)
    w1p = w1p.at[_Y_OFF:_Y_OFF + _NUM_CLASSES].set(
        jnp.dot(emb, w1y, preferred_element_type=f32))
    w1p = w1p.at[_T_OFF:_T_OFF + _TIME_EMB].set(
        jnp.dot(wt2, w1t, preferred_element_type=f32))
    b1c = (b1 + jnp.dot(bt2, w1t, preferred_element_type=f32)).reshape(-1, 1)
    w4p = jnp.zeros((w4.shape[0], _OUT_ROWS), f32).at[:, :latent_dim].set(w4)
    b4c = jnp.zeros((_OUT_ROWS, 1), f32).at[:latent_dim].set(b4.reshape(-1, 1))

    weights = (wt1.reshape(-1, 1), bt1.reshape(-1, 1),
               w1p.astype(bf16), b1c,
               w2.astype(bf16), b2.reshape(-1, 1),
               w3.astype(bf16), b3.reshape(-1, 1),
               w4p.astype(bf16), b4c)

    VMEM = pltpu.MemorySpace.VMEM

    def const_spec(shape):
        return pl.BlockSpec(shape, lambda i: (0, 0), memory_space=VMEM)

    in_specs = [pl.BlockSpec((_C_ROWS, block_n), lambda i: (0, i),
                             memory_space=VMEM)]
    in_specs += [const_spec(w.shape) for w in weights]

    out_t = pl.pallas_call(
        _mlp_kernel,
        out_shape=jax.ShapeDtypeStruct((_OUT_ROWS, Bp), f32),
        grid=(Bp // block_n,),
        in_specs=in_specs,
        out_specs=pl.BlockSpec((_OUT_ROWS, block_n), lambda i: (0, i),
                               memory_space=VMEM),
        scratch_shapes=[pltpu.VMEM((_SLAB_K, block_n), bf16)],
        compiler_params=pltpu.CompilerParams(
            dimension_semantics=("parallel",)),
    )(c, *weights)
    return out_t[:latent_dim, :B].T


# speedup vs baseline: 5.2205x; 1.0610x over previous
import jax
import jax.numpy as jnp
from jax.experimental import pallas as pl
from jax.experimental.pallas import tpu as pltpu

_LATENT = 10
_NUM_CLASSES = 10
_TIME_EMB = 32
_TIMESTEPS = 300
_SLAB_K = 64
_X_OFF = 0
_Y_OFF = 16
_T_OFF = 32
_C_ROWS = 16
_TN_ROW = 10
_Y_ROW = 11
_OUT_ROWS = 16


def _round_up(n, m):
    return ((n + m - 1) // m) * m


def _mlp_kernel(c_ref,
                wt1c_ref, bt1c_ref,
                w1p_ref, b1c_ref, w2_ref, b2c_ref,
                w3_ref, b3c_ref, w4p_ref, b4c_ref,
                out_ref, slab_ref):
    f32 = jnp.float32
    bf16 = jnp.bfloat16
    bn = c_ref.shape[1]

    tn_row = c_ref[_TN_ROW:_TN_ROW + 1, :]
    y_row = c_ref[_Y_ROW:_Y_ROW + 1, :].astype(jnp.int32)

    sub16 = jax.lax.broadcasted_iota(jnp.int32, (_C_ROWS, bn), 0)
    slab_ref[_X_OFF:_X_OFF + _C_ROWS, :] = jnp.where(
        sub16 < _LATENT, c_ref[...], 0.0).astype(bf16)
    slab_ref[_Y_OFF:_Y_OFF + _C_ROWS, :] = (sub16 == y_row).astype(bf16)
    slab_ref[_T_OFF:_T_OFF + _TIME_EMB, :] = jnp.maximum(
        wt1c_ref[...] * tn_row + bt1c_ref[...], 0.0).astype(bf16)

    dn = (((0,), (0,)), ((), ()))
    h1 = jnp.maximum(
        jax.lax.dot_general(w1p_ref[...], slab_ref[...], dn,
                            preferred_element_type=f32) + b1c_ref[...], 0.0)
    h2 = jnp.maximum(
        jax.lax.dot_general(w2_ref[...], h1.astype(bf16), dn,
                            preferred_element_type=f32) + b2c_ref[...], 0.0)
    h3 = jnp.maximum(
        jax.lax.dot_general(w3_ref[...], h2.astype(bf16), dn,
                            preferred_element_type=f32) + b3c_ref[...], 0.0)
    out_ref[...] = jax.lax.dot_general(w4p_ref[...], h3.astype(bf16), dn,
                                       preferred_element_type=f32) + b4c_ref[...]


def kernel(emb, wt1, bt1, wt2, bt2, w1, b1, w2, b2, w3, b3, w4, b4, x, t, y):
    f32 = jnp.float32
    bf16 = jnp.bfloat16
    B, latent_dim = x.shape

    block_n = 4096 if B >= 4096 else max(128, _round_up(B, 128))
    Bp = _round_up(B, block_n)

    c = jnp.concatenate([
        x.T,
        (t.astype(f32) / _TIMESTEPS).reshape(1, B),
        y.astype(f32).reshape(1, B),
        jnp.zeros((_C_ROWS - latent_dim - 2, B), f32),
    ])
    if Bp != B:
        c = jnp.pad(c, ((0, 0), (0, Bp - B)))

    w1x = w1[:latent_dim]
    w1t = w1[latent_dim:latent_dim + _TIME_EMB]
    w1y = w1[latent_dim + _TIME_EMB:]
    w1p = jnp.zeros((_SLAB_K, w1.shape[1]), f32)
    w1p = w1p.at[_X_OFF:_X_OFF + latent_dim].set(w1x)
    w1p = w1p.at[_Y_OFF:_Y_OFF + _NUM_CLASSES].set(
        jnp.dot(emb, w1y, preferred_element_type=f32))
    w1p = w1p.at[_T_OFF:_T_OFF + _TIME_EMB].set(
        jnp.dot(wt2, w1t, preferred_element_type=f32))
    b1c = (b1 + jnp.dot(bt2, w1t, preferred_element_type=f32)).reshape(-1, 1)
    w4p = jnp.zeros((w4.shape[0], _OUT_ROWS), f32).at[:, :latent_dim].set(w4)
    b4c = jnp.zeros((_OUT_ROWS, 1), f32).at[:latent_dim].set(b4.reshape(-1, 1))

    weights = (wt1.reshape(-1, 1), bt1.reshape(-1, 1),
               w1p.astype(bf16), b1c,
               w2.astype(bf16), b2.reshape(-1, 1),
               w3.astype(bf16), b3.reshape(-1, 1),
               w4p.astype(bf16), b4c)

    VMEM = pltpu.MemorySpace.VMEM

    def const_spec(shape):
        return pl.BlockSpec(shape, lambda i: (0, 0), memory_space=VMEM)

    in_specs = [pl.BlockSpec((_C_ROWS, block_n), lambda i: (0, i),
                             memory_space=VMEM)]
    in_specs += [const_spec(w.shape) for w in weights]

    out_t = pl.pallas_call(
        _mlp_kernel,
        out_shape=jax.ShapeDtypeStruct((_OUT_ROWS, Bp), f32),
        grid=(Bp // block_n,),
        in_specs=in_specs,
        out_specs=pl.BlockSpec((_OUT_ROWS, block_n), lambda i: (0, i),
                               memory_space=VMEM),
        scratch_shapes=[pltpu.VMEM((_SLAB_K, block_n), bf16)],
        compiler_params=pltpu.CompilerParams(
            dimension_semantics=("parallel",)),
    )(c, *weights)
    return out_t[:latent_dim, :B].T


# speedup vs baseline: 5.4080x; 1.0359x over previous
import jax
import jax.numpy as jnp
from jax.experimental import pallas as pl
from jax.experimental.pallas import tpu as pltpu

_LATENT = 10
_NUM_CLASSES = 10
_TIME_EMB = 32
_TIMESTEPS = 300
_SLAB_K = 64
_X_OFF = 0
_Y_OFF = 16
_T_OFF = 32
_C_ROWS = 16
_TN_ROW = 10
_Y_ROW = 11
_OUT_ROWS = 16


def _round_up(n, m):
    return ((n + m - 1) // m) * m


def _mlp_kernel(c_ref,
                wt1c_ref, bt1c_ref,
                w1p_ref, b1c_ref, w2_ref, b2c_ref,
                w3_ref, b3c_ref, w4p_ref, b4c_ref,
                out_ref, slab_ref):
    f32 = jnp.float32
    bf16 = jnp.bfloat16
    bn = c_ref.shape[1]

    tn_row = c_ref[_TN_ROW:_TN_ROW + 1, :]
    y_row = c_ref[_Y_ROW:_Y_ROW + 1, :].astype(jnp.int32)

    sub16 = jax.lax.broadcasted_iota(jnp.int32, (_C_ROWS, bn), 0)
    slab_ref[_X_OFF:_X_OFF + _C_ROWS, :] = jnp.where(
        sub16 < _LATENT, c_ref[...], 0.0).astype(bf16)
    slab_ref[_Y_OFF:_Y_OFF + _C_ROWS, :] = (sub16 == y_row).astype(bf16)
    slab_ref[_T_OFF:_T_OFF + _TIME_EMB, :] = jnp.maximum(
        wt1c_ref[...] * tn_row + bt1c_ref[...], 0.0).astype(bf16)

    dn = (((0,), (0,)), ((), ()))
    h1 = jnp.maximum(
        jax.lax.dot_general(w1p_ref[...], slab_ref[...], dn,
                            preferred_element_type=f32) + b1c_ref[...], 0.0)
    h2 = jnp.maximum(
        jax.lax.dot_general(w2_ref[...], h1.astype(bf16), dn,
                            preferred_element_type=f32) + b2c_ref[...], 0.0)
    h3 = jnp.maximum(
        jax.lax.dot_general(w3_ref[...], h2.astype(bf16), dn,
                            preferred_element_type=f32) + b3c_ref[...], 0.0)
    out_ref[...] = jax.lax.dot_general(w4p_ref[...], h3.astype(bf16), dn,
                                       preferred_element_type=f32) + b4c_ref[...]


def kernel(emb, wt1, bt1, wt2, bt2, w1, b1, w2, b2, w3, b3, w4, b4, x, t, y):
    f32 = jnp.float32
    bf16 = jnp.bfloat16
    B, latent_dim = x.shape

    block_n = 8192 if B >= 8192 else max(128, _round_up(B, 128))
    Bp = _round_up(B, block_n)

    c = jnp.concatenate([
        x.T,
        (t.astype(f32) / _TIMESTEPS).reshape(1, B),
        y.astype(f32).reshape(1, B),
        jnp.zeros((_C_ROWS - latent_dim - 2, B), f32),
    ])
    if Bp != B:
        c = jnp.pad(c, ((0, 0), (0, Bp - B)))

    w1x = w1[:latent_dim]
    w1t = w1[latent_dim:latent_dim + _TIME_EMB]
    w1y = w1[latent_dim + _TIME_EMB:]
    w1p = jnp.zeros((_SLAB_K, w1.shape[1]), f32)
    w1p = w1p.at[_X_OFF:_X_OFF + latent_dim].set(w1x)
    w1p = w1p.at[_Y_OFF:_Y_OFF + _NUM_CLASSES].set(
        jnp.dot(emb, w1y, preferred_element_type=f32))
    w1p = w1p.at[_T_OFF:_T_OFF + _TIME_EMB].set(
        jnp.dot(wt2, w1t, preferred_element_type=f32))
    b1c = (b1 + jnp.dot(bt2, w1t, preferred_element_type=f32)).reshape(-1, 1)
    w4p = jnp.zeros((w4.shape[0], _OUT_ROWS), f32).at[:, :latent_dim].set(w4)
    b4c = jnp.zeros((_OUT_ROWS, 1), f32).at[:latent_dim].set(b4.reshape(-1, 1))

    weights = (wt1.reshape(-1, 1), bt1.reshape(-1, 1),
               w1p.astype(bf16), b1c,
               w2.astype(bf16), b2.reshape(-1, 1),
               w3.astype(bf16), b3.reshape(-1, 1),
               w4p.astype(bf16), b4c)

    VMEM = pltpu.MemorySpace.VMEM

    def const_spec(shape):
        return pl.BlockSpec(shape, lambda i: (0, 0), memory_space=VMEM)

    in_specs = [pl.BlockSpec((_C_ROWS, block_n), lambda i: (0, i),
                             memory_space=VMEM)]
    in_specs += [const_spec(w.shape) for w in weights]

    out_t = pl.pallas_call(
        _mlp_kernel,
        out_shape=jax.ShapeDtypeStruct((_OUT_ROWS, Bp), f32),
        grid=(Bp // block_n,),
        in_specs=in_specs,
        out_specs=pl.BlockSpec((_OUT_ROWS, block_n), lambda i: (0, i),
                               memory_space=VMEM),
        scratch_shapes=[pltpu.VMEM((_SLAB_K, block_n), bf16)],
        compiler_params=pltpu.CompilerParams(
            dimension_semantics=("parallel",)),
    )(c, *weights)
    return out_t[:latent_dim, :B].T


# speedup vs baseline: 5.4311x; 1.0043x over previous
import jax
import jax.numpy as jnp
from jax.experimental import pallas as pl
from jax.experimental.pallas import tpu as pltpu

_LATENT = 10
_NUM_CLASSES = 10
_TIME_EMB = 32
_TIMESTEPS = 300
_SLAB_K = 64
_X_OFF = 0
_Y_OFF = 16
_T_OFF = 32
_C_ROWS = 16
_TN_ROW = 10
_Y_ROW = 11
_OUT_ROWS = 16


def _round_up(n, m):
    return ((n + m - 1) // m) * m


def _mlp_kernel(c_ref,
                wt1c_ref, bt1c_ref,
                w1p_ref, b1c_ref, w2_ref, b2c_ref,
                w3_ref, b3c_ref, w4p_ref, b4c_ref,
                out_ref, slab_ref):
    f32 = jnp.float32
    bf16 = jnp.bfloat16
    bn = c_ref.shape[1]

    tn_row = c_ref[_TN_ROW:_TN_ROW + 1, :]
    y_row = c_ref[_Y_ROW:_Y_ROW + 1, :].astype(jnp.int32)

    sub16 = jax.lax.broadcasted_iota(jnp.int32, (_C_ROWS, bn), 0)
    slab_ref[_X_OFF:_X_OFF + _C_ROWS, :] = jnp.where(
        sub16 < _LATENT, c_ref[...], 0.0).astype(bf16)
    slab_ref[_Y_OFF:_Y_OFF + _C_ROWS, :] = (sub16 == y_row).astype(bf16)
    slab_ref[_T_OFF:_T_OFF + _TIME_EMB, :] = jnp.maximum(
        wt1c_ref[...] * tn_row + bt1c_ref[...], 0.0).astype(bf16)

    dn = (((0,), (0,)), ((), ()))
    h1 = jnp.maximum(
        jax.lax.dot_general(w1p_ref[...], slab_ref[...], dn,
                            preferred_element_type=f32) + b1c_ref[...], 0.0)
    h2 = jnp.maximum(
        jax.lax.dot_general(w2_ref[...], h1.astype(bf16), dn,
                            preferred_element_type=f32) + b2c_ref[...], 0.0)
    h3 = jnp.maximum(
        jax.lax.dot_general(w3_ref[...], h2.astype(bf16), dn,
                            preferred_element_type=f32) + b3c_ref[...], 0.0)
    out_ref[...] = jax.lax.dot_general(w4p_ref[...], h3.astype(bf16), dn,
                                       preferred_element_type=f32) + b4c_ref[...]


def kernel(emb, wt1, bt1, wt2, bt2, w1, b1, w2, b2, w3, b3, w4, b4, x, t, y):
    f32 = jnp.float32
    bf16 = jnp.bfloat16
    B, latent_dim = x.shape

    block_n = 16384 if B >= 16384 else max(128, _round_up(B, 128))
    Bp = _round_up(B, block_n)

    c = jnp.concatenate([
        x.T,
        (t.astype(f32) / _TIMESTEPS).reshape(1, B),
        y.astype(f32).reshape(1, B),
        jnp.zeros((_C_ROWS - latent_dim - 2, B), f32),
    ])
    if Bp != B:
        c = jnp.pad(c, ((0, 0), (0, Bp - B)))

    w1x = w1[:latent_dim]
    w1t = w1[latent_dim:latent_dim + _TIME_EMB]
    w1y = w1[latent_dim + _TIME_EMB:]
    w1p = jnp.zeros((_SLAB_K, w1.shape[1]), f32)
    w1p = w1p.at[_X_OFF:_X_OFF + latent_dim].set(w1x)
    w1p = w1p.at[_Y_OFF:_Y_OFF + _NUM_CLASSES].set(
        jnp.dot(emb, w1y, preferred_element_type=f32))
    w1p = w1p.at[_T_OFF:_T_OFF + _TIME_EMB].set(
        jnp.dot(wt2, w1t, preferred_element_type=f32))
    b1c = (b1 + jnp.dot(bt2, w1t, preferred_element_type=f32)).reshape(-1, 1)
    w4p = jnp.zeros((w4.shape[0], _OUT_ROWS), f32).at[:, :latent_dim].set(w4)
    b4c = jnp.zeros((_OUT_ROWS, 1), f32).at[:latent_dim].set(b4.reshape(-1, 1))

    weights = (wt1.reshape(-1, 1), bt1.reshape(-1, 1),
               w1p.astype(bf16), b1c,
               w2.astype(bf16), b2.reshape(-1, 1),
               w3.astype(bf16), b3.reshape(-1, 1),
               w4p.astype(bf16), b4c)

    VMEM = pltpu.MemorySpace.VMEM

    def const_spec(shape):
        return pl.BlockSpec(shape, lambda i: (0, 0), memory_space=VMEM)

    in_specs = [pl.BlockSpec((_C_ROWS, block_n), lambda i: (0, i),
                             memory_space=VMEM)]
    in_specs += [const_spec(w.shape) for w in weights]

    out_t = pl.pallas_call(
        _mlp_kernel,
        out_shape=jax.ShapeDtypeStruct((_OUT_ROWS, Bp), f32),
        grid=(Bp // block_n,),
        in_specs=in_specs,
        out_specs=pl.BlockSpec((_OUT_ROWS, block_n), lambda i: (0, i),
                               memory_space=VMEM),
        scratch_shapes=[pltpu.VMEM((_SLAB_K, block_n), bf16)],
        compiler_params=pltpu.CompilerParams(
            dimension_semantics=("parallel",)),
    )(c, *weights)
    return out_t[:latent_dim, :B].T
